# speedup vs baseline: 1.9661x; 1.9661x over previous
import functools

import jax
import jax.numpy as jnp
from jax import lax
from jax.experimental import pallas as pl
from jax.experimental.pallas import tpu as pltpu
from jax.experimental.pallas import tpu_sc as plsc

VOCAB_N = 50257
D = 1024
BATCH = 4
SEQ = 2048
NTOK = BATCH * SEQ
LN_EPS = 1e-05

NC = 2
NS = 16
NW = NC * NS
RPW = NTOK // NW
CH = 32
NCH = RPW // CH


def _sc_gather(table, idx2d):
    mesh = plsc.VectorSubcoreMesh(core_axis_name="c", subcore_axis_name="s")

    @functools.partial(
        pl.kernel,
        mesh=mesh,
        out_type=jax.ShapeDtypeStruct((NTOK, D), jnp.float32),
        scratch_types=[
            pltpu.VMEM((NCH, CH), jnp.int32),
            pltpu.VMEM((CH, D), jnp.float32),
            pltpu.SemaphoreType.DMA,
        ],
    )
    def k(table_hbm, idx_hbm, out_hbm, idx_v, buf, sem):
        wid = lax.axis_index("s") * NC + lax.axis_index("c")
        pltpu.sync_copy(idx_hbm.at[pl.ds(wid * NCH, NCH)], idx_v)
        base = wid * RPW
        for c in range(NCH):
            pltpu.async_copy(table_hbm.at[idx_v.at[c]], buf, sem).wait()
            pltpu.sync_copy(buf, out_hbm.at[pl.ds(base + c * CH, CH)])

    return k(table, idx2d)


SB = 256


def _tc_body(emb_ref, invf_ref, w_ref, b_ref, out_ref):
    i = pl.program_id(0)
    t = (i * SB + lax.broadcasted_iota(jnp.int32, (SB, 1), 0)).astype(jnp.float32)
    freq = t * invf_ref[...]
    cos_v = jnp.cos(freq)
    sin_v = jnp.sin(freq)
    x = emb_ref[...]
    xr = jnp.roll(x, 1, axis=-1)
    y = x * cos_v[None] + xr * sin_v[None]
    mu = jnp.mean(y, axis=-1, keepdims=True)
    yc = y - mu
    var = jnp.mean(yc * yc, axis=-1, keepdims=True)
    out = yc * lax.rsqrt(var + LN_EPS) * w_ref[...] + b_ref[...]
    out_ref[0] = out


def _tc_rotary_ln(emb3, invf, lnw, lnb, interpret=False):
    return pl.pallas_call(
        _tc_body,
        grid=(SEQ // SB,),
        in_specs=[
            pl.BlockSpec((BATCH, SB, D), lambda i: (0, i, 0)),
            pl.BlockSpec((1, D), lambda i: (0, 0)),
            pl.BlockSpec((1, D), lambda i: (0, 0)),
            pl.BlockSpec((1, D), lambda i: (0, 0)),
        ],
        out_specs=pl.BlockSpec((1, BATCH, SB, D), lambda i: (0, 0, i, 0)),
        out_shape=jax.ShapeDtypeStruct((1, BATCH, SEQ, D), jnp.float32),
        interpret=interpret,
    )(emb3, invf.reshape(1, D), lnw.reshape(1, D), lnb.reshape(1, D))


def kernel(input_ids, token_emb, ln_w, ln_b):
    ids = input_ids.reshape(-1).astype(jnp.int32)
    emb = _sc_gather(token_emb, ids.reshape(NTOK // CH, CH))
    invf_half = 1.0 / (10000.0 ** (jnp.arange(0, D, 2, dtype=jnp.float32) / D))
    invf = jnp.concatenate([invf_half, invf_half])
    return _tc_rotary_ln(emb.reshape(BATCH, SEQ, D), invf, ln_w, ln_b)


# speedup vs baseline: 2.4478x; 1.2450x over previous
import functools

import jax
import jax.numpy as jnp
import numpy as np
from jax import lax
from jax.experimental import pallas as pl
from jax.experimental.pallas import tpu as pltpu
from jax.experimental.pallas import tpu_sc as plsc

VOCAB_N = 50257
D = 1024
BATCH = 4
SEQ = 2048
NTOK = BATCH * SEQ
LN_EPS = 1e-05

NC = 2
NS = 16
NW = NC * NS
RPW = NTOK // NW
CH = 32
NCH = RPW // CH


def _sc_gather(table, idx2d):
    mesh = plsc.VectorSubcoreMesh(core_axis_name="c", subcore_axis_name="s")

    @functools.partial(
        pl.kernel,
        mesh=mesh,
        out_type=jax.ShapeDtypeStruct((NTOK, D), jnp.float32),
        scratch_types=[
            pltpu.VMEM((NCH, CH), jnp.int32),
            pltpu.VMEM((CH, D), jnp.float32),
            pltpu.SemaphoreType.DMA,
        ],
    )
    def k(table_hbm, idx_hbm, out_hbm, idx_v, buf, sem):
        wid = lax.axis_index("s") * NC + lax.axis_index("c")
        pltpu.sync_copy(idx_hbm.at[pl.ds(wid * NCH, NCH)], idx_v)
        base = wid * RPW
        for c in range(NCH):
            pltpu.async_copy(table_hbm.at[idx_v.at[c]], buf, sem).wait()
            pltpu.sync_copy(buf, out_hbm.at[pl.ds(base + c * CH, CH)])

    return k(table, idx2d)


SB = 256

_FREQ_HALF = np.arange(SEQ, dtype=np.float32)[:, None] * (
    1.0 / (10000.0 ** (np.arange(0, D, 2, dtype=np.float32) / np.float32(D)))
)[None, :]
_COS_HALF = np.cos(_FREQ_HALF)
_SIN_HALF = np.sin(_FREQ_HALF)


def _tc_body(emb_ref, cos_ref, sin_ref, w_ref, b_ref, out_ref):
    cos_v = jnp.concatenate([cos_ref[...], cos_ref[...]], axis=-1)
    sin_v = jnp.concatenate([sin_ref[...], sin_ref[...]], axis=-1)
    x = emb_ref[...]
    xr = pltpu.roll(x, 1, 2)
    y = x * cos_v[None] + xr * sin_v[None]
    mu = jnp.mean(y, axis=-1, keepdims=True)
    yc = y - mu
    var = jnp.mean(yc * yc, axis=-1, keepdims=True)
    out = yc * lax.rsqrt(var + LN_EPS) * w_ref[...] + b_ref[...]
    out_ref[0] = out


def _tc_rotary_ln(emb3, cos_h, sin_h, lnw, lnb, interpret=False):
    return pl.pallas_call(
        _tc_body,
        grid=(SEQ // SB,),
        in_specs=[
            pl.BlockSpec((BATCH, SB, D), lambda i: (0, i, 0)),
            pl.BlockSpec((SB, D // 2), lambda i: (i, 0)),
            pl.BlockSpec((SB, D // 2), lambda i: (i, 0)),
            pl.BlockSpec((1, D), lambda i: (0, 0)),
            pl.BlockSpec((1, D), lambda i: (0, 0)),
        ],
        out_specs=pl.BlockSpec((1, BATCH, SB, D), lambda i: (0, 0, i, 0)),
        out_shape=jax.ShapeDtypeStruct((1, BATCH, SEQ, D), jnp.float32),
        interpret=interpret,
    )(emb3, cos_h, sin_h, lnw.reshape(1, D), lnb.reshape(1, D))


def kernel(input_ids, token_emb, ln_w, ln_b):
    ids = input_ids.reshape(-1).astype(jnp.int32)
    emb = _sc_gather(token_emb, ids.reshape(NTOK // CH, CH))
    cos_h = jnp.asarray(_COS_HALF)
    sin_h = jnp.asarray(_SIN_HALF)
    return _tc_rotary_ln(emb.reshape(BATCH, SEQ, D), cos_h, sin_h, ln_w, ln_b)


# speedup vs baseline: 2.6590x; 1.0863x over previous
import functools

import jax
import jax.numpy as jnp
import numpy as np
from jax import lax
from jax.experimental import pallas as pl
from jax.experimental.pallas import tpu as pltpu
from jax.experimental.pallas import tpu_sc as plsc

VOCAB_N = 50257
D = 1024
BATCH = 4
SEQ = 2048
NTOK = BATCH * SEQ
LN_EPS = 1e-05

NC = 2
NS = 16
NW = NC * NS
RPW = NTOK // NW
CH = 32
NCH = RPW // CH


def _sc_gather(table, idx2d):
    mesh = plsc.VectorSubcoreMesh(core_axis_name="c", subcore_axis_name="s")

    @functools.partial(
        pl.kernel,
        mesh=mesh,
        out_type=jax.ShapeDtypeStruct((NTOK, D), jnp.float32),
        scratch_types=[
            pltpu.VMEM((NCH, CH), jnp.int32),
            pltpu.VMEM((CH, D), jnp.float32),
            pltpu.VMEM((CH, D), jnp.float32),
            pltpu.VMEM((CH, D), jnp.float32),
            pltpu.SemaphoreType.DMA,
            pltpu.SemaphoreType.DMA,
            pltpu.SemaphoreType.DMA,
            pltpu.SemaphoreType.DMA,
            pltpu.SemaphoreType.DMA,
            pltpu.SemaphoreType.DMA,
        ],
    )
    def k(table_hbm, idx_hbm, out_hbm, idx_v, b0, b1, b2,
          gs0, gs1, gs2, ws0, ws1, ws2):
        bufs = (b0, b1, b2)
        gsems = (gs0, gs1, gs2)
        wsems = (ws0, ws1, ws2)
        wid = lax.axis_index("s") * NC + lax.axis_index("c")
        pltpu.sync_copy(idx_hbm.at[pl.ds(wid * NCH, NCH)], idx_v)
        base = wid * RPW
        g = [None] * NCH
        w = [None] * NCH
        for c in range(min(3, NCH)):
            g[c] = pltpu.async_copy(
                table_hbm.at[idx_v.at[c]], bufs[c % 3], gsems[c % 3])
        for c in range(NCH):
            k3 = c % 3
            g[c].wait()
            w[c] = pltpu.async_copy(
                bufs[k3], out_hbm.at[pl.ds(base + c * CH, CH)], wsems[k3])
            nxt = c + 3
            if nxt < NCH:
                w[c].wait()
                g[nxt] = pltpu.async_copy(
                    table_hbm.at[idx_v.at[nxt]], bufs[k3], gsems[k3])
        for c in range(max(0, NCH - 3), NCH):
            w[c].wait()

    return k(table, idx2d)


SB = 256

_FREQ_HALF = np.arange(SEQ, dtype=np.float32)[:, None] * (
    1.0 / (10000.0 ** (np.arange(0, D, 2, dtype=np.float32) / np.float32(D)))
)[None, :]
_COS_HALF = np.cos(_FREQ_HALF)
_SIN_HALF = np.sin(_FREQ_HALF)


def _tc_body(emb_ref, cos_ref, sin_ref, w_ref, b_ref, out_ref):
    cos_v = jnp.concatenate([cos_ref[...], cos_ref[...]], axis=-1)
    sin_v = jnp.concatenate([sin_ref[...], sin_ref[...]], axis=-1)
    x = emb_ref[...]
    xr = pltpu.roll(x, 1, 2)
    y = x * cos_v[None] + xr * sin_v[None]
    mu = jnp.mean(y, axis=-1, keepdims=True)
    yc = y - mu
    var = jnp.mean(yc * yc, axis=-1, keepdims=True)
    out = yc * lax.rsqrt(var + LN_EPS) * w_ref[...] + b_ref[...]
    out_ref[0] = out


def _tc_rotary_ln(emb3, cos_h, sin_h, lnw, lnb, interpret=False):
    return pl.pallas_call(
        _tc_body,
        grid=(SEQ // SB,),
        in_specs=[
            pl.BlockSpec((BATCH, SB, D), lambda i: (0, i, 0)),
            pl.BlockSpec((SB, D // 2), lambda i: (i, 0)),
            pl.BlockSpec((SB, D // 2), lambda i: (i, 0)),
            pl.BlockSpec((1, D), lambda i: (0, 0)),
            pl.BlockSpec((1, D), lambda i: (0, 0)),
        ],
        out_specs=pl.BlockSpec((1, BATCH, SB, D), lambda i: (0, 0, i, 0)),
        out_shape=jax.ShapeDtypeStruct((1, BATCH, SEQ, D), jnp.float32),
        interpret=interpret,
    )(emb3, cos_h, sin_h, lnw.reshape(1, D), lnb.reshape(1, D))


def kernel(input_ids, token_emb, ln_w, ln_b):
    ids = input_ids.reshape(-1).astype(jnp.int32)
    emb = _sc_gather(token_emb, ids.reshape(NTOK // CH, CH))
    cos_h = jnp.asarray(_COS_HALF)
    sin_h = jnp.asarray(_SIN_HALF)
    return _tc_rotary_ln(emb.reshape(BATCH, SEQ, D), cos_h, sin_h, ln_w, ln_b)
